# Initial kernel scaffold; baseline (speedup 1.0000x reference)
import jax
import jax.numpy as jnp
from jax.experimental import pallas as pl


def kernel(t, embed_table):
    raise NotImplementedError("write your pallas kernel here")


# speedup vs baseline: 9.1922x; 9.1922x over previous
import functools

import jax
import jax.numpy as jnp
from jax import lax
from jax.experimental import pallas as pl
from jax.experimental.pallas import tpu as pltpu
from jax.experimental.pallas import tpu_sc as plsc

D = 128
NC = 2
NS = 16
NW = NC * NS
CHUNK = 128
NSLOT = 5
LA = 2


@functools.lru_cache(maxsize=None)
def _gather_call(n_chunks):
    assert n_chunks % NSLOT == 0 and n_chunks >= NSLOT
    b_per_w = n_chunks * CHUNK
    B = NW * b_per_w
    n_outer = n_chunks // NSLOT
    mesh = plsc.VectorSubcoreMesh(core_axis_name="c", subcore_axis_name="s")

    @functools.partial(
        pl.kernel,
        mesh=mesh,
        out_type=jax.ShapeDtypeStruct((B, D), jnp.float32),
        scratch_types=(
            [
                pltpu.VMEM((n_chunks, CHUNK), jnp.int32),
                pltpu.VMEM((NSLOT, CHUNK, D), jnp.float32),
            ]
            + [pltpu.SemaphoreType.DMA] * (1 + 2 * NSLOT)
        ),
    )
    def k(table_hbm, idx_hbm, out_hbm, idx_v, rows_v, *sems):
        isem = sems[0]
        gsems = sems[1 : 1 + NSLOT]
        osems = sems[1 + NSLOT :]

        wid = lax.axis_index("s") * NC + lax.axis_index("c")
        base = wid * b_per_w

        pltpu.async_copy(idx_hbm.at[wid], idx_v, isem).wait()

        def g_start(j, slot):
            pltpu.async_copy(table_hbm.at[idx_v.at[j]], rows_v.at[slot],
                             gsems[slot])

        def g_wait(j, slot):
            pltpu.make_async_copy(table_hbm.at[idx_v.at[j]], rows_v.at[slot],
                                  gsems[slot]).wait()

        def w_start(j, slot):
            pltpu.async_copy(rows_v.at[slot],
                             out_hbm.at[pl.ds(base + j * CHUNK, CHUNK)],
                             osems[slot])

        def w_wait(j, slot):
            pltpu.make_async_copy(rows_v.at[slot],
                                  out_hbm.at[pl.ds(base + j * CHUNK, CHUNK)],
                                  osems[slot]).wait()

        for b in range(LA):
            g_start(b, b)

        def outer(g, carry):
            j0 = g * NSLOT
            for b in range(NSLOT):
                j = j0 + b
                g_wait(j, b)
                w_start(j, b)
                ns = (b + LA) % NSLOT

                @pl.when(j + LA < n_chunks)
                def _():
                    @pl.when(j + LA >= NSLOT)
                    def _():
                        w_wait(j + LA - NSLOT, ns)

                    g_start(j + LA, ns)

            return carry

        lax.fori_loop(0, n_outer, outer, 0)

        for j in range(n_chunks - NSLOT, n_chunks):
            w_wait(j, j % NSLOT)

    return k


def kernel(t, embed_table):
    assert t.ndim == 2, "Expects (B, T)"
    Bt, T = t.shape
    total = Bt * T
    assert total % (NW * CHUNK) == 0
    n_chunks = total // (NW * CHUNK)
    idx = t.reshape(NW, n_chunks, CHUNK).astype(jnp.int32)
    out = _gather_call(n_chunks)(embed_table, idx)
    return out.reshape(Bt, T, D)
